# Initial kernel scaffold; baseline (speedup 1.0000x reference)
import jax
import jax.numpy as jnp
from jax.experimental import pallas as pl


def kernel(data, edge_index, batch, params):
    raise NotImplementedError("write your pallas kernel here")


# speedup vs baseline: 4.3148x; 4.3148x over previous
import functools

import jax
import jax.numpy as jnp
from jax import lax
from jax.experimental import pallas as pl
from jax.experimental.pallas import tpu as pltpu
from jax.experimental.pallas import tpu_sc as plsc

N = 10000
NP = 10240
E = 320000
D = 128
G = 256
NC, NS = 2, 16
NW = NC * NS
EPT = 10240
EP = EPT * NW
CH = 128
NCHUNK = EPT // CH
RPT = NP // NS
EPS_LN = 1e-5


def _sc_mesh():
    return plsc.VectorSubcoreMesh(
        core_axis_name="c", subcore_axis_name="s",
        num_cores=NC, num_subcores=NS)


def _sc_agg_body(x_hbm, src_hbm, dst_hbm, zeros_hbm, out_hbm,
                 idx_s, idx_d, rows, accum, sem):
    cid = lax.axis_index("c")
    sid = lax.axis_index("s")
    pltpu.sync_copy(zeros_hbm, accum.at[pl.ds(sid * RPT, RPT)])
    plsc.subcore_barrier()
    base = cid * (NS * EPT) + sid * EPT

    def body(c, _):
        off = base + c * CH
        pltpu.sync_copy(src_hbm.at[pl.ds(off, CH)], idx_s.at[0])
        pltpu.sync_copy(dst_hbm.at[pl.ds(off, CH)], idx_d.at[0])
        pltpu.async_copy(x_hbm.at[idx_s.at[0]], rows, sem).wait()
        pltpu.sync_copy(rows, accum.at[idx_d.at[0]], add=True)
        return ()

    lax.fori_loop(0, NCHUNK, body, (), unroll=False)
    plsc.subcore_barrier()
    pltpu.sync_copy(accum.at[pl.ds(sid * RPT, RPT)],
                    out_hbm.at[cid, pl.ds(sid * RPT, RPT)])


def _sc_agg(x, srcp, dstp, zeros2d):
    return pl.kernel(
        _sc_agg_body,
        out_type=jax.ShapeDtypeStruct((NC, NP, D), jnp.float32),
        mesh=_sc_mesh(),
        scratch_types=[
            pltpu.VMEM((1, CH), jnp.int32),
            pltpu.VMEM((1, CH), jnp.int32),
            pltpu.VMEM((CH, D), jnp.float32),
            pltpu.VMEM_SHARED((NP, D), jnp.float32),
            pltpu.SemaphoreType.DMA,
        ],
    )(x, srcp, dstp, zeros2d)


def _sc_scal_body(s_hbm, src_hbm, dst_hbm, zeros_hbm, out_hbm,
                  idx_s, idx_d, vals, accum, sem):
    cid = lax.axis_index("c")
    sid = lax.axis_index("s")
    pltpu.sync_copy(zeros_hbm, accum.at[pl.ds(sid * RPT, RPT)])
    plsc.subcore_barrier()
    base = cid * (NS * EPT) + sid * EPT

    def body(c, _):
        off = base + c * CH
        pltpu.sync_copy(src_hbm.at[pl.ds(off, CH)], idx_s.at[0])
        pltpu.sync_copy(dst_hbm.at[pl.ds(off, CH)], idx_d.at[0])
        pltpu.async_copy(s_hbm.at[idx_s.at[0]], vals, sem).wait()
        pltpu.sync_copy(vals, accum.at[idx_d.at[0]], add=True)
        return ()

    lax.fori_loop(0, NCHUNK, body, (), unroll=False)
    plsc.subcore_barrier()
    pltpu.sync_copy(accum.at[pl.ds(sid * RPT, RPT)],
                    out_hbm.at[cid, pl.ds(sid * RPT, RPT)])


def _sc_scal(s, srcp, dstp, zeros1d):
    return pl.kernel(
        _sc_scal_body,
        out_type=jax.ShapeDtypeStruct((NC, NP), jnp.float32),
        mesh=_sc_mesh(),
        scratch_types=[
            pltpu.VMEM((1, CH), jnp.int32),
            pltpu.VMEM((1, CH), jnp.int32),
            pltpu.VMEM((CH,), jnp.float32),
            pltpu.VMEM_SHARED((NP,), jnp.float32),
            pltpu.SemaphoreType.DMA,
        ],
    )(s, srcp, dstp, zeros1d)


def _seg_sum(onehot, v):
    return lax.dot_general(onehot, v, (((0,), (0,)), ((), ())),
                           preferred_element_type=jnp.float32)


def _tc_a_body(x_ref, parts_ref, batch_ref, w1_ref, b1_ref, w2_ref, b2_ref,
               wrel_ref, lnw_ref, lnb_ref, x2_ref, srel_ref, xn_ref):
    x = x_ref[...]
    agg = parts_ref[0] + parts_ref[1]
    h = jnp.dot(x + agg, w1_ref[...], preferred_element_type=jnp.float32)
    h = jnp.maximum(h + b1_ref[...][None, :], 0.0)
    x2 = jnp.dot(h, w2_ref[...], preferred_element_type=jnp.float32)
    x2 = x2 + b2_ref[...][None, :]
    rows = lax.broadcasted_iota(jnp.int32, (NP, 1), 0)
    valid = (rows < N).astype(jnp.float32)
    x2 = x2 * valid
    x2_ref[...] = x2
    srel_ref[...] = jnp.dot(x2, wrel_ref[...],
                            preferred_element_type=jnp.float32) * valid
    b = batch_ref[...]
    onehot = (b == lax.broadcasted_iota(jnp.int32, (NP, G), 1)
              ).astype(jnp.float32)
    deg = _seg_sum(onehot, valid)
    norm = jnp.maximum(deg, 1.0) * D
    rowsum = jnp.sum(x2, axis=1, keepdims=True)
    mean = _seg_sum(onehot, rowsum) / norm
    mean_n = jnp.dot(onehot, mean, preferred_element_type=jnp.float32)
    xc = x2 - mean_n
    sq = jnp.sum(xc * xc, axis=1, keepdims=True)
    var = _seg_sum(onehot, sq) / norm
    rstd_n = jnp.dot(onehot, 1.0 / jnp.sqrt(var + EPS_LN),
                     preferred_element_type=jnp.float32)
    xn = xc * rstd_n * lnw_ref[...][None, :] + lnb_ref[...][None, :]
    xn_ref[...] = jnp.maximum(xn, 0.0) * valid


def _tc_a(x, parts, batch_pad, p):
    return pl.pallas_call(
        _tc_a_body,
        out_shape=(
            jax.ShapeDtypeStruct((NP, D), jnp.float32),
            jax.ShapeDtypeStruct((NP, 1), jnp.float32),
            jax.ShapeDtypeStruct((NP, D), jnp.float32),
        ),
    )(x, parts, batch_pad, p['w1'], p['b1'], p['w2'], p['b2'],
      p['w_rel'], p['ln_w'], p['ln_b'])


def _tc_b_body(x2_ref, e_ref, batch_ref, wroot_ref, brel_ref, g_ref):
    x2 = x2_ref[...]
    sroot = jnp.dot(x2, wroot_ref[...], preferred_element_type=jnp.float32)
    score = e_ref[0] + e_ref[1] + brel_ref[...] + sroot
    b = batch_ref[...]
    onehot_b = (b == lax.broadcasted_iota(jnp.int32, (NP, G), 1))
    onehot = onehot_b.astype(jnp.float32)
    scm = jnp.where(onehot_b, score, -1e30)
    smax = jnp.max(scm, axis=0, keepdims=True)
    smax_n = lax.dot_general(onehot, smax, (((1,), (1,)), ((), ())),
                             preferred_element_type=jnp.float32)
    rows = lax.broadcasted_iota(jnp.int32, (NP, 1), 0)
    ee = jnp.where(rows < N, jnp.exp(score - smax_n), 0.0)
    denom = _seg_sum(onehot, ee)
    denom_n = jnp.dot(onehot, denom, preferred_element_type=jnp.float32)
    sm = ee / (denom_n + 1e-16)
    g_ref[...] = _seg_sum(onehot, x2 * sm)


def _tc_b(x2, eparts, batch_pad, p):
    return pl.pallas_call(
        _tc_b_body,
        out_shape=jax.ShapeDtypeStruct((G, D), jnp.float32),
    )(x2, eparts.reshape(NC, NP, 1), batch_pad, p['w_root'],
      p['b_rel'].reshape(1, 1))


def kernel(data, edge_index, batch, params):
    src = edge_index[0].astype(jnp.int32)
    dst = edge_index[1].astype(jnp.int32)
    pad_e = jnp.full((EP - E,), N, dtype=jnp.int32)
    srcp = jnp.concatenate([src, pad_e])
    dstp = jnp.concatenate([dst, pad_e])
    x = jnp.pad(data, ((0, NP - N), (0, 0)))
    batch_pad = jnp.concatenate(
        [batch.astype(jnp.int32),
         jnp.full((NP - N,), G, dtype=jnp.int32)]).reshape(NP, 1)
    zeros2d = jnp.zeros((RPT, D), jnp.float32)
    zeros1d = jnp.zeros((RPT,), jnp.float32)

    embs = []
    for p in params:
        parts = _sc_agg(x, srcp, dstp, zeros2d)
        x2, srel, xn = _tc_a(x, parts, batch_pad, p)
        eparts = _sc_scal(srel[:, 0], srcp, dstp, zeros1d)
        g = _tc_b(x2, eparts, batch_pad, p)
        embs.append(g)
        x = xn
    return jnp.stack(embs, axis=1)


# speedup vs baseline: 6.1641x; 1.4286x over previous
import functools

import jax
import jax.numpy as jnp
from jax import lax
from jax.experimental import pallas as pl
from jax.experimental.pallas import tpu as pltpu
from jax.experimental.pallas import tpu_sc as plsc

N = 10000
NP = 10240
E = 320000
D = 128
G = 256
NC, NS = 2, 16
NW = NC * NS
EPT = 10240
EP = EPT * NW
CH = 128
NCHUNK = EPT // CH
RPT = NP // NS
EPS_LN = 1e-5


def _sc_mesh():
    return plsc.VectorSubcoreMesh(
        core_axis_name="c", subcore_axis_name="s",
        num_cores=NC, num_subcores=NS)


def _sc_agg_body(x_hbm, src_hbm, dst_hbm, zeros_hbm, out_hbm,
                 src_slab, idx_d, rows0, rows1, accum,
                 semi, semd0, semd1, sem0, sem1):
    cid = lax.axis_index("c")
    sid = lax.axis_index("s")
    eb = (cid * NS + sid) * EPT
    pltpu.async_copy(src_hbm.at[pl.ds(eb, EPT)], src_slab, semi)
    pltpu.async_copy(dst_hbm.at[pl.ds(eb, CH)], idx_d.at[0], semd0)
    pltpu.async_copy(dst_hbm.at[pl.ds(eb + CH, CH)], idx_d.at[1], semd1)
    pltpu.sync_copy(zeros_hbm, accum.at[pl.ds(sid * RPT, RPT)])
    pltpu.make_async_copy(src_hbm.at[pl.ds(eb, EPT)], src_slab, semi).wait()
    plsc.subcore_barrier()
    pltpu.async_copy(x_hbm.at[src_slab.at[pl.ds(0, CH)]], rows0, sem0)
    pltpu.async_copy(x_hbm.at[src_slab.at[pl.ds(CH, CH)]], rows1, sem1)

    def step(c, rows, sem, db, semd):
        pltpu.make_async_copy(x_hbm.at[src_slab.at[pl.ds(0, CH)]],
                              rows, sem).wait()
        pltpu.make_async_copy(dst_hbm.at[pl.ds(eb, CH)], db, semd).wait()
        pltpu.sync_copy(rows, accum.at[db], add=True)
        pltpu.async_copy(
            x_hbm.at[src_slab.at[pl.ds((c + 2) * CH, CH)]], rows, sem)
        pltpu.async_copy(dst_hbm.at[pl.ds(eb + (c + 2) * CH, CH)], db, semd)

    def body(i, _):
        c = 2 * i
        step(c, rows0, sem0, idx_d.at[0], semd0)
        step(c + 1, rows1, sem1, idx_d.at[1], semd1)
        return ()

    lax.fori_loop(0, NCHUNK // 2 - 1, body, (), unroll=False)
    c = NCHUNK - 2
    pltpu.make_async_copy(x_hbm.at[src_slab.at[pl.ds(0, CH)]],
                          rows0, sem0).wait()
    pltpu.make_async_copy(dst_hbm.at[pl.ds(eb, CH)], idx_d.at[0],
                          semd0).wait()
    pltpu.sync_copy(rows0, accum.at[idx_d.at[0]], add=True)
    pltpu.make_async_copy(x_hbm.at[src_slab.at[pl.ds(0, CH)]],
                          rows1, sem1).wait()
    pltpu.make_async_copy(dst_hbm.at[pl.ds(eb, CH)], idx_d.at[1],
                          semd1).wait()
    pltpu.sync_copy(rows1, accum.at[idx_d.at[1]], add=True)
    plsc.subcore_barrier()
    pltpu.sync_copy(accum.at[pl.ds(sid * RPT, RPT)],
                    out_hbm.at[cid, pl.ds(sid * RPT, RPT)])


def _sc_agg(x, srcp, dstp, zeros2d):
    return pl.kernel(
        _sc_agg_body,
        out_type=jax.ShapeDtypeStruct((NC, NP, D), jnp.float32),
        mesh=_sc_mesh(),
        scratch_types=[
            pltpu.VMEM((EPT,), jnp.int32),
            pltpu.VMEM((2, CH), jnp.int32),
            pltpu.VMEM((CH, D), jnp.float32),
            pltpu.VMEM((CH, D), jnp.float32),
            pltpu.VMEM_SHARED((NP, D), jnp.float32),
            pltpu.SemaphoreType.DMA,
            pltpu.SemaphoreType.DMA,
            pltpu.SemaphoreType.DMA,
            pltpu.SemaphoreType.DMA,
            pltpu.SemaphoreType.DMA,
        ],
    )(x, srcp, dstp, zeros2d)


def _sc_scal_body(s_hbm, src_hbm, dst_hbm, zeros_hbm, out_hbm,
                  src_slab, idx_d, vals0, vals1, accum,
                  semi, semd0, semd1, sem0, sem1):
    cid = lax.axis_index("c")
    sid = lax.axis_index("s")
    eb = (cid * NS + sid) * EPT
    pltpu.async_copy(src_hbm.at[pl.ds(eb, EPT)], src_slab, semi)
    pltpu.async_copy(dst_hbm.at[pl.ds(eb, CH)], idx_d.at[0], semd0)
    pltpu.async_copy(dst_hbm.at[pl.ds(eb + CH, CH)], idx_d.at[1], semd1)
    pltpu.sync_copy(zeros_hbm, accum.at[pl.ds(sid * RPT, RPT)])
    pltpu.make_async_copy(src_hbm.at[pl.ds(eb, EPT)], src_slab, semi).wait()
    plsc.subcore_barrier()
    pltpu.async_copy(s_hbm.at[src_slab.at[pl.ds(0, CH)]], vals0, sem0)
    pltpu.async_copy(s_hbm.at[src_slab.at[pl.ds(CH, CH)]], vals1, sem1)

    def step(c, vals, sem, db, semd):
        pltpu.make_async_copy(s_hbm.at[src_slab.at[pl.ds(0, CH)]],
                              vals, sem).wait()
        pltpu.make_async_copy(dst_hbm.at[pl.ds(eb, CH)], db, semd).wait()
        pltpu.sync_copy(vals, accum.at[db], add=True)
        pltpu.async_copy(
            s_hbm.at[src_slab.at[pl.ds((c + 2) * CH, CH)]], vals, sem)
        pltpu.async_copy(dst_hbm.at[pl.ds(eb + (c + 2) * CH, CH)], db, semd)

    def body(i, _):
        c = 2 * i
        step(c, vals0, sem0, idx_d.at[0], semd0)
        step(c + 1, vals1, sem1, idx_d.at[1], semd1)
        return ()

    lax.fori_loop(0, NCHUNK // 2 - 1, body, (), unroll=False)
    pltpu.make_async_copy(s_hbm.at[src_slab.at[pl.ds(0, CH)]],
                          vals0, sem0).wait()
    pltpu.make_async_copy(dst_hbm.at[pl.ds(eb, CH)], idx_d.at[0],
                          semd0).wait()
    pltpu.sync_copy(vals0, accum.at[idx_d.at[0]], add=True)
    pltpu.make_async_copy(s_hbm.at[src_slab.at[pl.ds(0, CH)]],
                          vals1, sem1).wait()
    pltpu.make_async_copy(dst_hbm.at[pl.ds(eb, CH)], idx_d.at[1],
                          semd1).wait()
    pltpu.sync_copy(vals1, accum.at[idx_d.at[1]], add=True)
    plsc.subcore_barrier()
    pltpu.sync_copy(accum.at[pl.ds(sid * RPT, RPT)],
                    out_hbm.at[cid, pl.ds(sid * RPT, RPT)])


def _sc_scal(s, srcp, dstp, zeros1d):
    return pl.kernel(
        _sc_scal_body,
        out_type=jax.ShapeDtypeStruct((NC, NP), jnp.float32),
        mesh=_sc_mesh(),
        scratch_types=[
            pltpu.VMEM((EPT,), jnp.int32),
            pltpu.VMEM((2, CH), jnp.int32),
            pltpu.VMEM((CH,), jnp.float32),
            pltpu.VMEM((CH,), jnp.float32),
            pltpu.VMEM_SHARED((NP,), jnp.float32),
            pltpu.SemaphoreType.DMA,
            pltpu.SemaphoreType.DMA,
            pltpu.SemaphoreType.DMA,
            pltpu.SemaphoreType.DMA,
            pltpu.SemaphoreType.DMA,
        ],
    )(s, srcp, dstp, zeros1d)


def _seg_sum(onehot, v):
    return lax.dot_general(onehot, v, (((0,), (0,)), ((), ())),
                           preferred_element_type=jnp.float32)


def _tc_a_body(x_ref, parts_ref, batch_ref, w1_ref, b1_ref, w2_ref, b2_ref,
               wrel_ref, lnw_ref, lnb_ref, x2_ref, srel_ref, xn_ref):
    x = x_ref[...]
    agg = parts_ref[0] + parts_ref[1]
    h = jnp.dot(x + agg, w1_ref[...], preferred_element_type=jnp.float32)
    h = jnp.maximum(h + b1_ref[...][None, :], 0.0)
    x2 = jnp.dot(h, w2_ref[...], preferred_element_type=jnp.float32)
    x2 = x2 + b2_ref[...][None, :]
    rows = lax.broadcasted_iota(jnp.int32, (NP, 1), 0)
    valid = (rows < N).astype(jnp.float32)
    x2 = x2 * valid
    x2_ref[...] = x2
    srel_ref[...] = jnp.dot(x2, wrel_ref[...],
                            preferred_element_type=jnp.float32) * valid
    b = batch_ref[...]
    onehot = (b == lax.broadcasted_iota(jnp.int32, (NP, G), 1)
              ).astype(jnp.float32)
    deg = _seg_sum(onehot, valid)
    norm = jnp.maximum(deg, 1.0) * D
    rowsum = jnp.sum(x2, axis=1, keepdims=True)
    mean = _seg_sum(onehot, rowsum) / norm
    mean_n = jnp.dot(onehot, mean, preferred_element_type=jnp.float32)
    xc = x2 - mean_n
    sq = jnp.sum(xc * xc, axis=1, keepdims=True)
    var = _seg_sum(onehot, sq) / norm
    rstd_n = jnp.dot(onehot, 1.0 / jnp.sqrt(var + EPS_LN),
                     preferred_element_type=jnp.float32)
    xn = xc * rstd_n * lnw_ref[...][None, :] + lnb_ref[...][None, :]
    xn_ref[...] = jnp.maximum(xn, 0.0) * valid


def _tc_a(x, parts, batch_pad, p):
    return pl.pallas_call(
        _tc_a_body,
        out_shape=(
            jax.ShapeDtypeStruct((NP, D), jnp.float32),
            jax.ShapeDtypeStruct((NP, 1), jnp.float32),
            jax.ShapeDtypeStruct((NP, D), jnp.float32),
        ),
    )(x, parts, batch_pad, p['w1'], p['b1'], p['w2'], p['b2'],
      p['w_rel'], p['ln_w'], p['ln_b'])


def _tc_b_body(x2_ref, e_ref, batch_ref, wroot_ref, brel_ref, g_ref):
    x2 = x2_ref[...]
    sroot = jnp.dot(x2, wroot_ref[...], preferred_element_type=jnp.float32)
    score = e_ref[0] + e_ref[1] + brel_ref[...] + sroot
    b = batch_ref[...]
    onehot_b = (b == lax.broadcasted_iota(jnp.int32, (NP, G), 1))
    onehot = onehot_b.astype(jnp.float32)
    scm = jnp.where(onehot_b, score, -1e30)
    smax = jnp.max(scm, axis=0, keepdims=True)
    smax_n = lax.dot_general(onehot, smax, (((1,), (1,)), ((), ())),
                             preferred_element_type=jnp.float32)
    rows = lax.broadcasted_iota(jnp.int32, (NP, 1), 0)
    ee = jnp.where(rows < N, jnp.exp(score - smax_n), 0.0)
    denom = _seg_sum(onehot, ee)
    denom_n = jnp.dot(onehot, denom, preferred_element_type=jnp.float32)
    sm = ee / (denom_n + 1e-16)
    g_ref[...] = _seg_sum(onehot, x2 * sm)


def _tc_b(x2, eparts, batch_pad, p):
    return pl.pallas_call(
        _tc_b_body,
        out_shape=jax.ShapeDtypeStruct((G, D), jnp.float32),
    )(x2, eparts.reshape(NC, NP, 1), batch_pad, p['w_root'],
      p['b_rel'].reshape(1, 1))


def kernel(data, edge_index, batch, params):
    src = edge_index[0].astype(jnp.int32)
    dst = edge_index[1].astype(jnp.int32)
    pad_e = jnp.full((EP - E,), N, dtype=jnp.int32)
    srcp = jnp.concatenate([src, pad_e])
    dstp = jnp.concatenate([dst, pad_e])
    x = jnp.pad(data, ((0, NP - N), (0, 0)))
    batch_pad = jnp.concatenate(
        [batch.astype(jnp.int32),
         jnp.full((NP - N,), G, dtype=jnp.int32)]).reshape(NP, 1)
    zeros2d = jnp.zeros((RPT, D), jnp.float32)
    zeros1d = jnp.zeros((RPT,), jnp.float32)

    embs = []
    for p in params:
        parts = _sc_agg(x, srcp, dstp, zeros2d)
        x2, srel, xn = _tc_a(x, parts, batch_pad, p)
        eparts = _sc_scal(srel[:, 0], srcp, dstp, zeros1d)
        g = _tc_b(x2, eparts, batch_pad, p)
        embs.append(g)
        x = xn
    return jnp.stack(embs, axis=1)
